# Initial kernel scaffold; baseline (speedup 1.0000x reference)
import jax
import jax.numpy as jnp
from jax.experimental import pallas as pl
from jax.experimental.pallas import tpu as pltpu

N, D, H, E = 8192, 1024, 512, 8

_HI = jax.lax.Precision.HIGHEST


def _dot(a, b, precision):
    return jax.lax.dot_general(
        a, b, (((1,), (0,)), ((), ())),
        precision=precision, preferred_element_type=jnp.float32)


def _bdot(a, b):
    return jnp.dot(a.astype(jnp.bfloat16), b.astype(jnp.bfloat16),
                   preferred_element_type=jnp.float32)


def _gate_kernel(x_ref, gW1_ref, gb1_ref, gW2_ref, gb2_ref, gW3_ref, gb3_ref,
                 fW1_ref, fb1_ref, fW2_ref, fb2_ref,
                 sW1_ref, sb1_ref, sW2_ref, sb2_ref,
                 probs_ref, w8_ref, fiber_ref, smooth_ref):
    x = x_ref[...]
    h = jnp.maximum(_dot(x, gW1_ref[...], _HI) + gb1_ref[...], 0.0)
    h = jnp.maximum(_dot(h, gW2_ref[...], _HI) + gb2_ref[...], 0.0)
    logits = _dot(h, gW3_ref[...], _HI) + gb3_ref[...]
    m = jnp.max(logits, axis=-1, keepdims=True)
    ex = jnp.exp(logits - m)
    p = ex / jnp.sum(ex, axis=-1, keepdims=True)
    probs_ref[...] = p

    col = jax.lax.broadcasted_iota(jnp.int32, p.shape, 1)
    v1 = jnp.max(p, axis=-1, keepdims=True)
    i1 = jnp.min(jnp.where(p == v1, col, E), axis=-1, keepdims=True)
    mask1 = col == i1
    pm = jnp.where(mask1, -1.0, p)
    v2 = jnp.max(pm, axis=-1, keepdims=True)
    i2 = jnp.min(jnp.where(pm == v2, col, E), axis=-1, keepdims=True)
    s = v1 + v2
    w8_ref[...] = (jnp.where(mask1, v1, 0.0)
                   + jnp.where(col == i2, v2, 0.0)) / s

    fh = jnp.maximum(_bdot(x, fW1_ref[...]) + fb1_ref[...], 0.0)
    fiber_ref[...] = jnp.tanh(_bdot(fh, fW2_ref[...]) + fb2_ref[...])
    sh = jnp.maximum(_bdot(x, sW1_ref[...]) + sb1_ref[...], 0.0)
    smooth_ref[...] = jax.nn.sigmoid(_bdot(sh, sW2_ref[...]) + sb2_ref[...])


def _expert_kernel(x_ref, w8_ref, eW1_ref, eb1_ref, eW2_ref, eb2_ref,
                   eW3_ref, eb3_ref, out_ref):
    e = pl.program_id(1)
    xb = x_ref[...]
    h = jnp.maximum(_bdot(xb, eW1_ref[0]) + eb1_ref[0], 0.0)
    h = jnp.maximum(_bdot(h, eW2_ref[0]) + eb2_ref[0], 0.0)
    o = _bdot(h, eW3_ref[0]) + eb3_ref[0]
    w = jax.lax.dynamic_slice_in_dim(w8_ref[...], e, 1, axis=1)
    contrib = o * w

    @pl.when(e == 0)
    def _():
        out_ref[...] = contrib

    @pl.when(e != 0)
    def _():
        out_ref[...] += contrib


def kernel(x, eW1, eb1, eW2, eb2, eW3, eb3, gW1, gb1, gW2, gb2, gW3, gb3,
           fW1, fb1, fW2, fb2, sW1, sb1, sW2, sb2):
    n = x.shape[0]
    TB = 1024
    SB = 2048

    r2 = lambda b: b.reshape(1, -1)
    gate_out = pl.pallas_call(
        _gate_kernel,
        grid=(n // TB,),
        in_specs=[
            pl.BlockSpec((TB, D), lambda i: (i, 0)),
            pl.BlockSpec((D, H), lambda i: (0, 0)),
            pl.BlockSpec((1, H), lambda i: (0, 0)),
            pl.BlockSpec((H, H), lambda i: (0, 0)),
            pl.BlockSpec((1, H), lambda i: (0, 0)),
            pl.BlockSpec((H, E), lambda i: (0, 0)),
            pl.BlockSpec((1, E), lambda i: (0, 0)),
            pl.BlockSpec((D, H), lambda i: (0, 0)),
            pl.BlockSpec((1, H), lambda i: (0, 0)),
            pl.BlockSpec((H, 1), lambda i: (0, 0)),
            pl.BlockSpec((1, 1), lambda i: (0, 0)),
            pl.BlockSpec((D, H), lambda i: (0, 0)),
            pl.BlockSpec((1, H), lambda i: (0, 0)),
            pl.BlockSpec((H, 1), lambda i: (0, 0)),
            pl.BlockSpec((1, 1), lambda i: (0, 0)),
        ],
        out_specs=[
            pl.BlockSpec((TB, E), lambda i: (i, 0)),
            pl.BlockSpec((TB, E), lambda i: (i, 0)),
            pl.BlockSpec((TB, 1), lambda i: (i, 0)),
            pl.BlockSpec((TB, 1), lambda i: (i, 0)),
        ],
        out_shape=[
            jax.ShapeDtypeStruct((n, E), jnp.float32),
            jax.ShapeDtypeStruct((n, E), jnp.float32),
            jax.ShapeDtypeStruct((n, 1), jnp.float32),
            jax.ShapeDtypeStruct((n, 1), jnp.float32),
        ],
        compiler_params=pltpu.CompilerParams(
            dimension_semantics=("parallel",)),
    )(x, gW1, r2(gb1), gW2, r2(gb2), gW3, r2(gb3),
      fW1, r2(fb1), fW2, r2(fb2), sW1, r2(sb1), sW2, r2(sb2))
    gate_probs, w8, fiber, smooth = gate_out

    out = pl.pallas_call(
        _expert_kernel,
        grid=(n // SB, E),
        in_specs=[
            pl.BlockSpec((SB, D), lambda s, e: (s, 0)),
            pl.BlockSpec((SB, E), lambda s, e: (s, 0)),
            pl.BlockSpec((1, D, H), lambda s, e: (e, 0, 0)),
            pl.BlockSpec((1, 1, H), lambda s, e: (e, 0, 0)),
            pl.BlockSpec((1, H, H), lambda s, e: (e, 0, 0)),
            pl.BlockSpec((1, 1, H), lambda s, e: (e, 0, 0)),
            pl.BlockSpec((1, H, D), lambda s, e: (e, 0, 0)),
            pl.BlockSpec((1, 1, D), lambda s, e: (e, 0, 0)),
        ],
        out_specs=pl.BlockSpec((SB, D), lambda s, e: (s, 0)),
        out_shape=jax.ShapeDtypeStruct((n, D), jnp.float32),
        compiler_params=pltpu.CompilerParams(
            dimension_semantics=("parallel", "arbitrary")),
    )(x, w8, eW1, eb1[:, None, :], eW2, eb2[:, None, :],
      eW3, eb3[:, None, :])

    return (out, gate_probs, fiber, smooth)


# speedup vs baseline: 1.3669x; 1.3669x over previous
import jax
import jax.numpy as jnp
from jax.experimental import pallas as pl
from jax.experimental.pallas import tpu as pltpu

N, D, H, E = 8192, 1024, 512, 8

_HI = jax.lax.Precision.HIGHEST


def _dot(a, b, precision):
    return jax.lax.dot_general(
        a, b, (((1,), (0,)), ((), ())),
        precision=precision, preferred_element_type=jnp.float32)


def _bdot(a, b):
    return jnp.dot(a.astype(jnp.bfloat16), b.astype(jnp.bfloat16),
                   preferred_element_type=jnp.float32)


def _gate_kernel(x_ref, gW1_ref, gb1_ref, gW2_ref, gb2_ref, gW3_ref, gb3_ref,
                 fW1_ref, fb1_ref, fW2_ref, fb2_ref,
                 sW1_ref, sb1_ref, sW2_ref, sb2_ref,
                 probs_ref, w8_ref, fiber_ref, smooth_ref):
    x = x_ref[...]
    h = jnp.maximum(_bdot(x, gW1_ref[...]) + gb1_ref[...], 0.0)
    h = jnp.maximum(_bdot(h, gW2_ref[...]) + gb2_ref[...], 0.0)
    logits = _bdot(h, gW3_ref[...]) + gb3_ref[...]
    m = jnp.max(logits, axis=-1, keepdims=True)
    ex = jnp.exp(logits - m)
    p = ex / jnp.sum(ex, axis=-1, keepdims=True)
    probs_ref[...] = p

    col = jax.lax.broadcasted_iota(jnp.int32, p.shape, 1)
    v1 = jnp.max(p, axis=-1, keepdims=True)
    i1 = jnp.min(jnp.where(p == v1, col, E), axis=-1, keepdims=True)
    mask1 = col == i1
    pm = jnp.where(mask1, -1.0, p)
    v2 = jnp.max(pm, axis=-1, keepdims=True)
    i2 = jnp.min(jnp.where(pm == v2, col, E), axis=-1, keepdims=True)
    s = v1 + v2
    w8_ref[...] = (jnp.where(mask1, v1, 0.0)
                   + jnp.where(col == i2, v2, 0.0)) / s

    fh = jnp.maximum(_bdot(x, fW1_ref[...]) + fb1_ref[...], 0.0)
    fiber_ref[...] = jnp.tanh(_bdot(fh, fW2_ref[...]) + fb2_ref[...])
    sh = jnp.maximum(_bdot(x, sW1_ref[...]) + sb1_ref[...], 0.0)
    smooth_ref[...] = jax.nn.sigmoid(_bdot(sh, sW2_ref[...]) + sb2_ref[...])


def _expert_kernel(x_ref, w8_ref, eW1_ref, eb1_ref, eW2_ref, eb2_ref,
                   eW3_ref, eb3_ref, out_ref):
    e = pl.program_id(1)
    xb = x_ref[...]
    h = jnp.maximum(_bdot(xb, eW1_ref[0]) + eb1_ref[0], 0.0)
    h = jnp.maximum(_bdot(h, eW2_ref[0]) + eb2_ref[0], 0.0)
    o = _bdot(h, eW3_ref[0]) + eb3_ref[0]
    w8v = w8_ref[...]
    col = jax.lax.broadcasted_iota(jnp.int32, w8v.shape, 1)
    w = jnp.sum(jnp.where(col == e, w8v, 0.0), axis=1, keepdims=True)
    contrib = o * w

    @pl.when(e == 0)
    def _():
        out_ref[...] = contrib

    @pl.when(e != 0)
    def _():
        out_ref[...] += contrib


def kernel(x, eW1, eb1, eW2, eb2, eW3, eb3, gW1, gb1, gW2, gb2, gW3, gb3,
           fW1, fb1, fW2, fb2, sW1, sb1, sW2, sb2):
    n = x.shape[0]
    TB = min(1024, n)
    SB = min(2048, n)

    r2 = lambda b: b.reshape(1, -1)
    gate_out = pl.pallas_call(
        _gate_kernel,
        grid=(n // TB,),
        in_specs=[
            pl.BlockSpec((TB, D), lambda i: (i, 0)),
            pl.BlockSpec((D, H), lambda i: (0, 0)),
            pl.BlockSpec((1, H), lambda i: (0, 0)),
            pl.BlockSpec((H, H), lambda i: (0, 0)),
            pl.BlockSpec((1, H), lambda i: (0, 0)),
            pl.BlockSpec((H, E), lambda i: (0, 0)),
            pl.BlockSpec((1, E), lambda i: (0, 0)),
            pl.BlockSpec((D, H), lambda i: (0, 0)),
            pl.BlockSpec((1, H), lambda i: (0, 0)),
            pl.BlockSpec((H, 1), lambda i: (0, 0)),
            pl.BlockSpec((1, 1), lambda i: (0, 0)),
            pl.BlockSpec((D, H), lambda i: (0, 0)),
            pl.BlockSpec((1, H), lambda i: (0, 0)),
            pl.BlockSpec((H, 1), lambda i: (0, 0)),
            pl.BlockSpec((1, 1), lambda i: (0, 0)),
        ],
        out_specs=[
            pl.BlockSpec((TB, E), lambda i: (i, 0)),
            pl.BlockSpec((TB, E), lambda i: (i, 0)),
            pl.BlockSpec((TB, 1), lambda i: (i, 0)),
            pl.BlockSpec((TB, 1), lambda i: (i, 0)),
        ],
        out_shape=[
            jax.ShapeDtypeStruct((n, E), jnp.float32),
            jax.ShapeDtypeStruct((n, E), jnp.float32),
            jax.ShapeDtypeStruct((n, 1), jnp.float32),
            jax.ShapeDtypeStruct((n, 1), jnp.float32),
        ],
        compiler_params=pltpu.CompilerParams(
            dimension_semantics=("parallel",)),
    )(x, gW1, r2(gb1), gW2, r2(gb2), gW3, r2(gb3),
      fW1, r2(fb1), fW2, r2(fb2), sW1, r2(sb1), sW2, r2(sb2))
    gate_probs, w8, fiber, smooth = gate_out

    out = pl.pallas_call(
        _expert_kernel,
        grid=(n // SB, E),
        in_specs=[
            pl.BlockSpec((SB, D), lambda s, e: (s, 0)),
            pl.BlockSpec((SB, E), lambda s, e: (s, 0)),
            pl.BlockSpec((1, D, H), lambda s, e: (e, 0, 0)),
            pl.BlockSpec((1, 1, H), lambda s, e: (e, 0, 0)),
            pl.BlockSpec((1, H, H), lambda s, e: (e, 0, 0)),
            pl.BlockSpec((1, 1, H), lambda s, e: (e, 0, 0)),
            pl.BlockSpec((1, H, D), lambda s, e: (e, 0, 0)),
            pl.BlockSpec((1, 1, D), lambda s, e: (e, 0, 0)),
        ],
        out_specs=pl.BlockSpec((SB, D), lambda s, e: (s, 0)),
        out_shape=jax.ShapeDtypeStruct((n, D), jnp.float32),
        compiler_params=pltpu.CompilerParams(
            dimension_semantics=("parallel", "arbitrary")),
    )(x, w8, eW1, eb1[:, None, :], eW2, eb2[:, None, :],
      eW3, eb3[:, None, :])

    return (out, gate_probs, fiber, smooth)
